# baseline (speedup 1.0000x reference)
import functools

import jax
import jax.numpy as jnp
from jax import lax
from jax.experimental import pallas as pl
from jax.experimental.pallas import tpu as pltpu
from jax.experimental.pallas import tpu_sc as plsc

_PREC = lax.Precision.DEFAULT


def _elu(v):
    return jnp.where(v > 0, v, jnp.exp(jnp.minimum(v, 0.0)) - 1.0)


def _l2n(u):
    ssq = jnp.sum(u * u, axis=1, keepdims=True)
    return u * lax.rsqrt(jnp.maximum(ssq, 1e-12))


_BR = 1000


def _tc_emb_call(x, W_emb, b_emb, Wp1_big):
    B, N, D = x.shape
    H = W_emb.shape[1]
    F = B * H

    def body(x_ref, we_ref, be_ref, wp_ref, h0_ref, p1_ref):
        we = we_ref[...]
        be = be_ref[...]
        hs = [_elu(jnp.dot(x_ref[bi], we, preferred_element_type=jnp.float32,
                           precision=_PREC) + be) for bi in range(B)]
        h0 = jnp.concatenate(hs, axis=1)
        h0_ref[...] = h0
        p1_ref[...] = _elu(jnp.dot(h0, wp_ref[...],
                                   preferred_element_type=jnp.float32,
                                   precision=_PREC))

    return pl.pallas_call(
        body,
        grid=(N // _BR,),
        in_specs=[
            pl.BlockSpec((B, _BR, D), lambda i: (0, i, 0)),
            pl.BlockSpec((D, H), lambda i: (0, 0)),
            pl.BlockSpec((1, H), lambda i: (0, 0)),
            pl.BlockSpec((F, F), lambda i: (0, 0)),
        ],
        out_specs=[pl.BlockSpec((_BR, F), lambda i: (i, 0)),
                   pl.BlockSpec((_BR, F), lambda i: (i, 0))],
        out_shape=[jax.ShapeDtypeStruct((N, F), jnp.float32),
                   jax.ShapeDtypeStruct((N, F), jnp.float32)],
    )(x, W_emb, b_emb, Wp1_big)


def _upd_l2n(h, agg, wu_big, bu2, H):
    z = jnp.concatenate([h, agg], axis=1)
    u = jnp.dot(z, wu_big, preferred_element_type=jnp.float32,
                precision=_PREC) + bu2
    usq = u * u
    outs = []
    for g in range(u.shape[1] // H):
        ug = u[:, g * H:(g + 1) * H]
        sg = jnp.sum(usq[:, g * H:(g + 1) * H], axis=1, keepdims=True)
        outs.append(ug * lax.rsqrt(jnp.maximum(sg, 1e-12)))
    return jnp.concatenate(outs, axis=1)


def _tc_upd_call(h_n, parts, Wu_big, bu2, Wp_big):
    N, F = h_n.shape
    H = F // 2

    def body(h_ref, pr_ref, wu_ref, bu_ref, wp_ref, h1_ref, p2_ref):
        agg = pr_ref[0] + pr_ref[1]
        un = _upd_l2n(h_ref[...], agg, wu_ref[...], bu_ref[...], H)
        h1_ref[...] = un
        p2_ref[...] = _elu(jnp.dot(un, wp_ref[...],
                                   preferred_element_type=jnp.float32,
                                   precision=_PREC))

    return pl.pallas_call(
        body,
        grid=(N // _BR,),
        in_specs=[
            pl.BlockSpec((_BR, F), lambda i: (i, 0)),
            pl.BlockSpec((2, _BR, F), lambda i: (0, i, 0)),
            pl.BlockSpec((2 * F, F), lambda i: (0, 0)),
            pl.BlockSpec((1, F), lambda i: (0, 0)),
            pl.BlockSpec((F, F), lambda i: (0, 0)),
        ],
        out_specs=[pl.BlockSpec((_BR, F), lambda i: (i, 0)),
                   pl.BlockSpec((_BR, F), lambda i: (i, 0))],
        out_shape=[jax.ShapeDtypeStruct((N, F), jnp.float32),
                   jax.ShapeDtypeStruct((N, F), jnp.float32)],
    )(h_n, parts, Wu_big, bu2, Wp_big)


def _tc_fin_call(h_n, parts, Wu_big, bu2, W_A, b_A, W_B, b_B):
    N, F = h_n.shape
    H = F // 2
    TA = W_A.shape[1]

    def body(h_ref, pr_ref, wu_ref, bu_ref, wa_ref, ba_ref, wb_ref, bb_ref,
             out_ref):
        agg = pr_ref[0] + pr_ref[1]
        h2 = _upd_l2n(h_ref[...], agg, wu_ref[...], bu_ref[...], H)
        t = _elu(jnp.dot(h2, wa_ref[...], preferred_element_type=jnp.float32,
                         precision=_PREC) + ba_ref[...])
        y = jnp.dot(t, wb_ref[...], preferred_element_type=jnp.float32,
                    precision=_PREC) + bb_ref[...]
        out_ref[...] = y

    return pl.pallas_call(
        body,
        grid=(N // _BR,),
        in_specs=[
            pl.BlockSpec((_BR, F), lambda i: (i, 0)),
            pl.BlockSpec((2, _BR, F), lambda i: (0, i, 0)),
            pl.BlockSpec((2 * F, F), lambda i: (0, 0)),
            pl.BlockSpec((1, F), lambda i: (0, 0)),
            pl.BlockSpec((F, TA), lambda i: (0, 0)),
            pl.BlockSpec((1, TA), lambda i: (0, 0)),
            pl.BlockSpec((TA, 8), lambda i: (0, 0)),
            pl.BlockSpec((1, 8), lambda i: (0, 0)),
        ],
        out_specs=pl.BlockSpec((_BR, 8), lambda i: (i, 0)),
        out_shape=jax.ShapeDtypeStruct((N, 8), jnp.float32),
    )(h_n, parts, Wu_big, bu2, W_A, b_A, W_B, b_B)


_CH = 80
_NBUF = 12


def _sc_agg_call(p_n, idx2):
    N, F = p_n.shape
    NC, NS = 2, 16
    _, NW, NCH, CH = idx2.shape
    assert NW == NC * NS
    RPT = -(-N // (8 * NS)) * 8
    NP = RPT * NS

    mesh = plsc.VectorSubcoreMesh(core_axis_name="c", subcore_axis_name="s")

    NB = _NBUF
    NQ = NCH // NB
    MAIN = NQ * NB
    ZR = 80

    def body(p_hbm, idx_hbm, out_hbm,
             pidx_v, nidx_v, rows_v, zero_v, acc_s, *sems):
        gsem = sems[:NB]
        ssem = sems[NB:]
        c = lax.axis_index("c")
        s = lax.axis_index("s")
        wid = s * NC + c

        def gstart(k, b):
            pltpu.async_copy(p_hbm.at[pidx_v.at[k]], rows_v.at[b], gsem[b])

        def gwait(k, b):
            pltpu.make_async_copy(p_hbm.at[pidx_v.at[k]], rows_v.at[b],
                                  gsem[b]).wait()

        def sstart(k, b):
            pltpu.async_copy(rows_v.at[b], acc_s.at[nidx_v.at[k]], ssem[b],
                             add=True)

        def swait(k, b):
            pltpu.make_async_copy(rows_v.at[b], acc_s.at[nidx_v.at[k]],
                                  ssem[b]).wait()

        pltpu.sync_copy(idx_hbm.at[0, wid], pidx_v)
        pltpu.sync_copy(idx_hbm.at[1, wid], nidx_v)

        for b in range(NB):
            gstart(b, b)

        def zb(i, _):
            for j in range(F // 16):
                zero_v[i, pl.ds(j * 16, 16)] = jnp.zeros((16,), jnp.float32)
            return 0
        lax.fori_loop(0, ZR, zb, 0)
        for j in range(RPT // ZR):
            pltpu.sync_copy(zero_v, acc_s.at[pl.ds(s * RPT + j * ZR, ZR)])
        if RPT % ZR:
            pltpu.sync_copy(zero_v.at[pl.ds(0, RPT % ZR)],
                            acc_s.at[pl.ds(s * RPT + (RPT // ZR) * ZR,
                                           RPT % ZR)])
        plsc.subcore_barrier()

        def quad(i, _):
            k0 = NB * i
            for b in range(NB):
                gwait(k0 + b, b)
                sstart(k0 + b, b)
            for b in range(NB):
                k = k0 + b

                @pl.when(k + NB < NCH)
                def _():
                    swait(k, b)
                    gstart(k + NB, b)
            return 0
        lax.fori_loop(0, NQ, quad, 0)
        for k in range(MAIN, NCH):
            gwait(k, k % NB)
            sstart(k, k % NB)
        for k in range(NCH - NB, NCH):
            swait(k, k % NB)
        plsc.subcore_barrier()

        pltpu.sync_copy(acc_s.at[pl.ds(s * RPT, RPT)],
                        out_hbm.at[c, pl.ds(s * RPT, RPT)])

    f = pl.kernel(
        body,
        out_type=jax.ShapeDtypeStruct((NC, NP, F), jnp.float32),
        mesh=mesh,
        compiler_params=pltpu.CompilerParams(use_tc_tiling_on_sc=False),
        scratch_types=[
            pltpu.VMEM((NCH, CH), jnp.int32),
            pltpu.VMEM((NCH, CH), jnp.int32),
            pltpu.VMEM((NB, CH, F), jnp.float32),
            pltpu.VMEM((ZR, F), jnp.float32),
            pltpu.VMEM_SHARED((NP, F), jnp.float32),
        ] + [pltpu.SemaphoreType.DMA] * (2 * NB),
    )
    return f(p_n, idx2)


def kernel(x, edges, edge_weights, W_emb, b_emb, W_prep1, W_upd1, b_upd1,
           W_prep2, W_upd2, b_upd2, W_y0a, b_y0a, W_y0b, b_y0b,
           W_y1a, b_y1a, W_y1b, b_y1b):
    B, N, D = x.shape
    E = edges.shape[0]
    H = W_emb.shape[1]

    NW = 32
    NS = 16
    RPT = -(-N // (8 * NS)) * 8
    NP = RPT * NS
    grp = NW * _CH
    E_pad = -(-E // grp) * grp
    pad_n = E_pad - E
    if pad_n:
        pad_src = (jnp.arange(pad_n, dtype=jnp.int32) * 997) % N
        pad_dst = N + (jnp.arange(pad_n, dtype=jnp.int32) % (NP - N))
        edges_p = jnp.concatenate(
            [edges.astype(jnp.int32),
             jnp.stack([pad_src, pad_dst], axis=1)], axis=0)
    else:
        edges_p = edges.astype(jnp.int32)
    idx2 = jnp.transpose(edges_p).reshape(2, NW, E_pad // (NW * _CH), _CH)

    F = B * H

    def blockdiag2(W):
        Z = jnp.zeros_like(W)
        return jnp.concatenate([jnp.concatenate([W, Z], axis=1),
                                jnp.concatenate([Z, W], axis=1)], axis=0)

    def upd_big(W_upd):
        return jnp.concatenate([blockdiag2(W_upd[:H]),
                                blockdiag2(W_upd[H:])], axis=0)

    Wp1_big = blockdiag2(W_prep1)
    Wp2_big = blockdiag2(W_prep2)
    Wu1_big = upd_big(W_upd1)
    Wu2_big = upd_big(W_upd2)
    bu1 = jnp.concatenate([b_upd1, b_upd1]).reshape(1, F)
    bu2 = jnp.concatenate([b_upd2, b_upd2]).reshape(1, F)

    HY = W_y0a.shape[1]
    Za = jnp.zeros_like(W_y0a)
    W_A = jnp.concatenate([
        jnp.concatenate([W_y0a, W_y1a, Za, Za], axis=1),
        jnp.concatenate([Za, Za, W_y0a, W_y1a], axis=1)], axis=0)
    b_A = jnp.concatenate([b_y0a, b_y1a, b_y0a, b_y1a]).reshape(1, 4 * HY)
    Zb = jnp.zeros_like(W_y0b)
    W_B = jnp.concatenate([
        jnp.concatenate([W_y0b, Zb, Zb, Zb], axis=1),
        jnp.concatenate([Zb, W_y1b, Zb, Zb], axis=1),
        jnp.concatenate([Zb, Zb, W_y0b, Zb], axis=1),
        jnp.concatenate([Zb, Zb, Zb, W_y1b], axis=1)], axis=0)
    W_B = jnp.concatenate([W_B, jnp.zeros((4 * HY, 4), jnp.float32)], axis=1)
    b_B = jnp.concatenate([b_y0b, b_y1b, b_y0b, b_y1b,
                           jnp.zeros((4,), jnp.float32)]).reshape(1, 8)

    h0_n, p1_n = _tc_emb_call(x, W_emb, b_emb.reshape(1, H), Wp1_big)
    parts1 = _sc_agg_call(p1_n, idx2)
    h1_n, p2_n = _tc_upd_call(h0_n, parts1, Wu1_big, bu1, Wp2_big)
    parts2 = _sc_agg_call(p2_n, idx2)
    y8 = _tc_fin_call(h1_n, parts2, Wu2_big, bu2, W_A, b_A, W_B, b_B)
    return y8[:, :4].reshape(N, B, 2).transpose(1, 0, 2)


# speedup vs baseline: 1.1299x; 1.1299x over previous
import functools

import jax
import jax.numpy as jnp
from jax import lax
from jax.experimental import pallas as pl
from jax.experimental.pallas import tpu as pltpu
from jax.experimental.pallas import tpu_sc as plsc

_PREC = lax.Precision.DEFAULT


def _elu(v):
    return jnp.where(v > 0, v, jnp.exp(jnp.minimum(v, 0.0)) - 1.0)


def _l2n(u):
    ssq = jnp.sum(u * u, axis=1, keepdims=True)
    return u * lax.rsqrt(jnp.maximum(ssq, 1e-12))


_BR = 2000


def _tc_emb_call(x, W_emb, b_emb, Wp1_big):
    B, N, D = x.shape
    H = W_emb.shape[1]
    F = B * H

    def body(x_ref, we_ref, be_ref, wp_ref, h0_ref, p1_ref):
        we = we_ref[...]
        be = be_ref[...]
        hs = [_elu(jnp.dot(x_ref[bi], we, preferred_element_type=jnp.float32,
                           precision=_PREC) + be) for bi in range(B)]
        h0 = jnp.concatenate(hs, axis=1)
        h0_ref[...] = h0
        p1_ref[...] = _elu(jnp.dot(h0, wp_ref[...],
                                   preferred_element_type=jnp.float32,
                                   precision=_PREC))

    return pl.pallas_call(
        body,
        grid=(N // _BR,),
        in_specs=[
            pl.BlockSpec((B, _BR, D), lambda i: (0, i, 0)),
            pl.BlockSpec((D, H), lambda i: (0, 0)),
            pl.BlockSpec((1, H), lambda i: (0, 0)),
            pl.BlockSpec((F, F), lambda i: (0, 0)),
        ],
        out_specs=[pl.BlockSpec((_BR, F), lambda i: (i, 0)),
                   pl.BlockSpec((_BR, F), lambda i: (i, 0))],
        out_shape=[jax.ShapeDtypeStruct((N, F), jnp.float32),
                   jax.ShapeDtypeStruct((N, F), jnp.float32)],
    )(x, W_emb, b_emb, Wp1_big)


def _upd_l2n(h, agg, wu_big, bu2, H):
    z = jnp.concatenate([h, agg], axis=1)
    u = jnp.dot(z, wu_big, preferred_element_type=jnp.float32,
                precision=_PREC) + bu2
    usq = u * u
    outs = []
    for g in range(u.shape[1] // H):
        ug = u[:, g * H:(g + 1) * H]
        sg = jnp.sum(usq[:, g * H:(g + 1) * H], axis=1, keepdims=True)
        outs.append(ug * lax.rsqrt(jnp.maximum(sg, 1e-12)))
    return jnp.concatenate(outs, axis=1)


def _tc_upd_call(h_n, parts, Wu_big, bu2, Wp_big):
    N, F = h_n.shape
    H = F // 2

    def body(h_ref, pr_ref, wu_ref, bu_ref, wp_ref, h1_ref, p2_ref):
        agg = pr_ref[0, :, :F] + pr_ref[1, :, :F]
        un = _upd_l2n(h_ref[...], agg, wu_ref[...], bu_ref[...], H)
        h1_ref[...] = un
        p2_ref[...] = _elu(jnp.dot(un, wp_ref[...],
                                   preferred_element_type=jnp.float32,
                                   precision=_PREC))

    return pl.pallas_call(
        body,
        grid=(N // _BR,),
        in_specs=[
            pl.BlockSpec((_BR, F), lambda i: (i, 0)),
            pl.BlockSpec((2, _BR, 128), lambda i: (0, i, 0)),
            pl.BlockSpec((2 * F, F), lambda i: (0, 0)),
            pl.BlockSpec((1, F), lambda i: (0, 0)),
            pl.BlockSpec((F, F), lambda i: (0, 0)),
        ],
        out_specs=[pl.BlockSpec((_BR, F), lambda i: (i, 0)),
                   pl.BlockSpec((_BR, F), lambda i: (i, 0))],
        out_shape=[jax.ShapeDtypeStruct((N, F), jnp.float32),
                   jax.ShapeDtypeStruct((N, F), jnp.float32)],
    )(h_n, parts, Wu_big, bu2, Wp_big)


def _tc_fin_call(h_n, parts, Wu_big, bu2, W_A, b_A, W_B, b_B):
    N, F = h_n.shape
    H = F // 2
    TA = W_A.shape[1]

    def body(h_ref, pr_ref, wu_ref, bu_ref, wa_ref, ba_ref, wb_ref, bb_ref,
             out_ref):
        agg = pr_ref[0, :, :F] + pr_ref[1, :, :F]
        h2 = _upd_l2n(h_ref[...], agg, wu_ref[...], bu_ref[...], H)
        t = _elu(jnp.dot(h2, wa_ref[...], preferred_element_type=jnp.float32,
                         precision=_PREC) + ba_ref[...])
        y = jnp.dot(t, wb_ref[...], preferred_element_type=jnp.float32,
                    precision=_PREC) + bb_ref[...]
        out_ref[...] = y

    return pl.pallas_call(
        body,
        grid=(N // _BR,),
        in_specs=[
            pl.BlockSpec((_BR, F), lambda i: (i, 0)),
            pl.BlockSpec((2, _BR, 128), lambda i: (0, i, 0)),
            pl.BlockSpec((2 * F, F), lambda i: (0, 0)),
            pl.BlockSpec((1, F), lambda i: (0, 0)),
            pl.BlockSpec((F, TA), lambda i: (0, 0)),
            pl.BlockSpec((1, TA), lambda i: (0, 0)),
            pl.BlockSpec((TA, 8), lambda i: (0, 0)),
            pl.BlockSpec((1, 8), lambda i: (0, 0)),
        ],
        out_specs=pl.BlockSpec((_BR, 8), lambda i: (i, 0)),
        out_shape=jax.ShapeDtypeStruct((N, 8), jnp.float32),
    )(h_n, parts, Wu_big, bu2, W_A, b_A, W_B, b_B)


_CH = 80
_NBUF = 12


def _sc_agg_call(p_n, idx2):
    N, F = p_n.shape
    NC, NS = 2, 16
    _, NW, NCH, CH = idx2.shape
    assert NW == NC * NS
    RPT = -(-N // (8 * NS)) * 8
    NP = RPT * NS

    mesh = plsc.VectorSubcoreMesh(core_axis_name="c", subcore_axis_name="s")

    NB = _NBUF
    NQ = NCH // NB
    MAIN = NQ * NB
    ZR = 80

    def body(p_hbm, idx_hbm, out_hbm,
             pidx_v, nidx_v, rows_v, zero_v, acc_s, *sems):
        gsem = sems[:NB]
        ssem = sems[NB:]
        c = lax.axis_index("c")
        s = lax.axis_index("s")
        wid = s * NC + c

        def gstart(k, b):
            pltpu.async_copy(p_hbm.at[pidx_v.at[k]], rows_v.at[b], gsem[b])

        def gwait(k, b):
            pltpu.make_async_copy(p_hbm.at[pidx_v.at[k]], rows_v.at[b],
                                  gsem[b]).wait()

        def sstart(k, b):
            pltpu.async_copy(rows_v.at[b], acc_s.at[nidx_v.at[k]], ssem[b],
                             add=True)

        def swait(k, b):
            pltpu.make_async_copy(rows_v.at[b], acc_s.at[nidx_v.at[k]],
                                  ssem[b]).wait()

        pltpu.sync_copy(idx_hbm.at[0, wid], pidx_v)
        pltpu.sync_copy(idx_hbm.at[1, wid], nidx_v)

        for b in range(NB):
            gstart(b, b)

        def zb(i, _):
            for j in range(F // 16):
                zero_v[i, pl.ds(j * 16, 16)] = jnp.zeros((16,), jnp.float32)
            return 0
        lax.fori_loop(0, ZR, zb, 0)
        for j in range(RPT // ZR):
            pltpu.sync_copy(zero_v, acc_s.at[pl.ds(s * RPT + j * ZR, ZR)])
        if RPT % ZR:
            pltpu.sync_copy(zero_v.at[pl.ds(0, RPT % ZR)],
                            acc_s.at[pl.ds(s * RPT + (RPT // ZR) * ZR,
                                           RPT % ZR)])
        plsc.subcore_barrier()

        def quad(i, _):
            k0 = NB * i
            for b in range(NB):
                gwait(k0 + b, b)
                sstart(k0 + b, b)
            for b in range(NB):
                k = k0 + b

                @pl.when(k + NB < NCH)
                def _():
                    swait(k, b)
                    gstart(k + NB, b)
            return 0
        lax.fori_loop(0, NQ, quad, 0)
        for k in range(MAIN, NCH):
            gwait(k, k % NB)
            sstart(k, k % NB)
        for k in range(NCH - NB, NCH):
            swait(k, k % NB)
        plsc.subcore_barrier()

        pltpu.sync_copy(acc_s.at[pl.ds(s * RPT, RPT)],
                        out_hbm.at[c, pl.ds(s * RPT, RPT), pl.ds(0, F)])

    f = pl.kernel(
        body,
        out_type=jax.ShapeDtypeStruct((NC, NP, 128), jnp.float32),
        mesh=mesh,
        compiler_params=pltpu.CompilerParams(use_tc_tiling_on_sc=False),
        scratch_types=[
            pltpu.VMEM((NCH, CH), jnp.int32),
            pltpu.VMEM((NCH, CH), jnp.int32),
            pltpu.VMEM((NB, CH, F), jnp.float32),
            pltpu.VMEM((ZR, F), jnp.float32),
            pltpu.VMEM_SHARED((NP, F), jnp.float32),
        ] + [pltpu.SemaphoreType.DMA] * (2 * NB),
    )
    return f(p_n, idx2)


def kernel(x, edges, edge_weights, W_emb, b_emb, W_prep1, W_upd1, b_upd1,
           W_prep2, W_upd2, b_upd2, W_y0a, b_y0a, W_y0b, b_y0b,
           W_y1a, b_y1a, W_y1b, b_y1b):
    B, N, D = x.shape
    E = edges.shape[0]
    H = W_emb.shape[1]

    NW = 32
    NS = 16
    RPT = -(-N // (8 * NS)) * 8
    NP = RPT * NS
    grp = NW * _CH
    E_pad = -(-E // grp) * grp
    pad_n = E_pad - E
    if pad_n:
        pad_src = (jnp.arange(pad_n, dtype=jnp.int32) * 997) % N
        pad_dst = N + (jnp.arange(pad_n, dtype=jnp.int32) % (NP - N))
        edges_p = jnp.concatenate(
            [edges.astype(jnp.int32),
             jnp.stack([pad_src, pad_dst], axis=1)], axis=0)
    else:
        edges_p = edges.astype(jnp.int32)
    idx2 = jnp.transpose(edges_p).reshape(2, NW, E_pad // (NW * _CH), _CH)

    F = B * H

    def blockdiag2(W):
        Z = jnp.zeros_like(W)
        return jnp.concatenate([jnp.concatenate([W, Z], axis=1),
                                jnp.concatenate([Z, W], axis=1)], axis=0)

    def upd_big(W_upd):
        return jnp.concatenate([blockdiag2(W_upd[:H]),
                                blockdiag2(W_upd[H:])], axis=0)

    Wp1_big = blockdiag2(W_prep1)
    Wp2_big = blockdiag2(W_prep2)
    Wu1_big = upd_big(W_upd1)
    Wu2_big = upd_big(W_upd2)
    bu1 = jnp.concatenate([b_upd1, b_upd1]).reshape(1, F)
    bu2 = jnp.concatenate([b_upd2, b_upd2]).reshape(1, F)

    HY = W_y0a.shape[1]
    Za = jnp.zeros_like(W_y0a)
    W_A = jnp.concatenate([
        jnp.concatenate([W_y0a, W_y1a, Za, Za], axis=1),
        jnp.concatenate([Za, Za, W_y0a, W_y1a], axis=1)], axis=0)
    b_A = jnp.concatenate([b_y0a, b_y1a, b_y0a, b_y1a]).reshape(1, 4 * HY)
    Zb = jnp.zeros_like(W_y0b)
    W_B = jnp.concatenate([
        jnp.concatenate([W_y0b, Zb, Zb, Zb], axis=1),
        jnp.concatenate([Zb, W_y1b, Zb, Zb], axis=1),
        jnp.concatenate([Zb, Zb, W_y0b, Zb], axis=1),
        jnp.concatenate([Zb, Zb, Zb, W_y1b], axis=1)], axis=0)
    W_B = jnp.concatenate([W_B, jnp.zeros((4 * HY, 4), jnp.float32)], axis=1)
    b_B = jnp.concatenate([b_y0b, b_y1b, b_y0b, b_y1b,
                           jnp.zeros((4,), jnp.float32)]).reshape(1, 8)

    h0_n, p1_n = _tc_emb_call(x, W_emb, b_emb.reshape(1, H), Wp1_big)
    parts1 = _sc_agg_call(p1_n, idx2)
    h1_n, p2_n = _tc_upd_call(h0_n, parts1, Wu1_big, bu1, Wp2_big)
    parts2 = _sc_agg_call(p2_n, idx2)
    y8 = _tc_fin_call(h1_n, parts2, Wu2_big, bu2, W_A, b_A, W_B, b_B)
    return y8[:, :4].reshape(N, B, 2).transpose(1, 0, 2)
